# Initial kernel scaffold; baseline (speedup 1.0000x reference)
import jax
import jax.numpy as jnp
from jax.experimental import pallas as pl


def kernel(x_num, x_cat, E, W1, b1, W2, b2, W3, b3, WA, bA, WB, bB):
    raise NotImplementedError("write your pallas kernel here")


# speedup vs baseline: 2.1703x; 2.1703x over previous
import functools

import jax
import jax.numpy as jnp
from jax import lax
from jax.experimental import pallas as pl
from jax.experimental.pallas import tpu as pltpu
from jax.experimental.pallas import tpu_sc as plsc

_NUM_CORES = 2
_NUM_SUBCORES = 16
_NW = _NUM_CORES * _NUM_SUBCORES


def _sc_gather(table, idx, chunk):
  n, = idx.shape
  _, d = table.shape
  per_w = n // _NW
  n_chunks = per_w // chunk
  assert per_w % chunk == 0 and chunk % 8 == 0

  mesh = plsc.VectorSubcoreMesh(core_axis_name="c", subcore_axis_name="s")

  @functools.partial(
      pl.kernel,
      out_type=jax.ShapeDtypeStruct((n, d), jnp.float32),
      mesh=mesh,
      scratch_types=[
          pltpu.VMEM((chunk,), jnp.int32),
          pltpu.VMEM((chunk, d), jnp.float32),
          pltpu.SemaphoreType.DMA,
      ],
      compiler_params=pltpu.CompilerParams(use_tc_tiling_on_sc=False),
  )
  def gather_kernel(table_hbm, idx_hbm, out_hbm, idx_v, rows_v, sem):
    wid = lax.axis_index("s") * _NUM_CORES + lax.axis_index("c")
    base = wid * per_w

    def body(g, carry):
      off = base + g * chunk
      pltpu.sync_copy(idx_hbm.at[pl.ds(off, chunk)], idx_v)
      pltpu.async_copy(table_hbm.at[idx_v], rows_v, sem).wait()
      pltpu.sync_copy(rows_v, out_hbm.at[pl.ds(off, chunk)])
      return carry

    lax.fori_loop(0, n_chunks, body, 0)

  return gather_kernel(table, idx)


def _tc_mlp(x_num, emb, w1n, w1e, b1, w2, b2, w3, b3, wab, bab, bm):
  b, nd = x_num.shape
  ed = emb.shape[1]
  grid = (b // bm,)

  def body(xn_ref, emb_ref, w1n_ref, w1e_ref, b1_ref, w2_ref, b2_ref,
           w3_ref, b3_ref, wab_ref, bab_ref, out_ref):
    h = jnp.dot(xn_ref[...], w1n_ref[...], preferred_element_type=jnp.float32)
    h = h + jnp.dot(emb_ref[...], w1e_ref[...],
                    preferred_element_type=jnp.float32)
    h = jnp.maximum(h + b1_ref[...], 0.0)
    h = jnp.maximum(
        jnp.dot(h, w2_ref[...], preferred_element_type=jnp.float32)
        + b2_ref[...], 0.0)
    h = jnp.maximum(
        jnp.dot(h, w3_ref[...], preferred_element_type=jnp.float32)
        + b3_ref[...], 0.0)
    out_ref[...] = (
        jnp.dot(h, wab_ref[...], preferred_element_type=jnp.float32)
        + bab_ref[...])

  full = lambda shape: pl.BlockSpec(shape, lambda i: (0, 0))
  return pl.pallas_call(
      body,
      grid=grid,
      in_specs=[
          pl.BlockSpec((bm, nd), lambda i: (i, 0)),
          pl.BlockSpec((bm, ed), lambda i: (i, 0)),
          full(w1n.shape),
          full(w1e.shape),
          full(b1.shape),
          full(w2.shape),
          full(b2.shape),
          full(w3.shape),
          full(b3.shape),
          full(wab.shape),
          full(bab.shape),
      ],
      out_specs=pl.BlockSpec((bm, 2), lambda i: (i, 0)),
      out_shape=jax.ShapeDtypeStruct((b, 2), jnp.float32),
  )(x_num, emb, w1n, w1e, b1, w2, b2, w3, b3, wab, bab)


def kernel(x_num, x_cat, E, W1, b1, W2, b2, W3, b3, WA, bA, WB, bB):
  f, v, d = E.shape
  b = x_cat.shape[0]
  nd = x_num.shape[1]

  table = E.reshape(f * v, d)
  idx = (x_cat + (jnp.arange(f, dtype=jnp.int32) * v)[None, :]).reshape(-1)
  emb = _sc_gather(table, idx, chunk=1664)
  emb = emb.reshape(b, f * d)

  wab = jnp.concatenate([WA, WB], axis=1)
  bab = jnp.concatenate([bA, bB])[None, :]
  out = _tc_mlp(x_num, emb, W1[:nd], W1[nd:], b1[None, :], W2, b2[None, :],
                W3, b3[None, :], wab, bab, bm=2048)
  return out[:, 0], out[:, 1]
